# baseline (speedup 1.0000x reference)
import functools

import jax
import jax.numpy as jnp
from jax import lax
from jax.experimental import pallas as pl
from jax.experimental.pallas import tpu as pltpu
from jax.experimental.pallas import tpu_sc as plsc

N_NODES = 10000
N_EDGES = 320000
ALPHA = 0.5

_F32 = jnp.float32
_EP = N_EDGES // 8


_NW = 32
_HE = N_EDGES // 2
_G_PER_W = 2 * _HE // _NW
_G_CHUNK = 1000
_G_STEPS = _G_PER_W // _G_CHUNK
_G_SRC_STEPS = _G_STEPS // 2

_S_PER_W = _HE // _NW
_S_CHUNK = 1000
_S_STEPS = _S_PER_W // _S_CHUNK
_S_IDXW = 100
_S_IDXROWS = _S_CHUNK // _S_IDXW

_sc_mesh = plsc.VectorSubcoreMesh(core_axis_name="c", subcore_axis_name="s")


def _make_gather(hx):
    half_w = _G_PER_W // 2

    @functools.partial(
        pl.kernel,
        mesh=_sc_mesh,
        out_type=jax.ShapeDtypeStruct((2 * _HE, 16), _F32),
        scratch_types=[
            pltpu.VMEM((3, _G_CHUNK), jnp.int32),
            pltpu.VMEM((3, _G_CHUNK, 16), _F32),
        ] + [pltpu.SemaphoreType.DMA] * 9,
        compiler_params=pltpu.CompilerParams(use_tc_tiling_on_sc=False),
    )
    def gather(table_hbm, idx_hbm, out_hbm, idx_v, rows_v,
               isem0, isem1, isem2, gsem0, gsem1, gsem2,
               wsem0, wsem1, wsem2):
        wid = lax.axis_index("s") * 2 + lax.axis_index("c")
        isems = (isem0, isem1, isem2)
        gsems = (gsem0, gsem1, gsem2)
        wsems = (wsem0, wsem1, wsem2)

        def bases(c):
            if c < _G_SRC_STEPS:
                off = wid * half_w + c * _G_CHUNK
                return hx * _HE + off, off
            off = wid * half_w + (c - _G_SRC_STEPS) * _G_CHUNK
            return N_EDGES + hx * _HE + off, _HE + off

        nbuf = 3
        idx_d = [None] * nbuf
        g_d = [None] * nbuf
        wb_d = [None] * nbuf
        for b in range(nbuf):
            idx_d[b] = pltpu.async_copy(
                idx_hbm.at[pl.ds(bases(b)[0], _G_CHUNK)], idx_v.at[b],
                isems[b])
        for b in range(nbuf - 1):
            idx_d[b].wait()
            g_d[b] = pltpu.async_copy(table_hbm.at[idx_v.at[b]],
                                      rows_v.at[b], gsems[b])
        for i in range(_G_STEPS):
            b = i % nbuf
            nxt = (i + nbuf - 1) % nbuf
            if i + nbuf - 1 < _G_STEPS:
                idx_d[nxt].wait()
                if wb_d[nxt] is not None:
                    wb_d[nxt].wait()
                g_d[nxt] = pltpu.async_copy(
                    table_hbm.at[idx_v.at[nxt]], rows_v.at[nxt], gsems[nxt])
            g_d[b].wait()
            if i + nbuf < _G_STEPS:
                idx_d[b] = pltpu.async_copy(
                    idx_hbm.at[pl.ds(bases(i + nbuf)[0], _G_CHUNK)],
                    idx_v.at[b], isems[b])
            wb_d[b] = pltpu.async_copy(
                rows_v.at[b], out_hbm.at[pl.ds(bases(i)[1], _G_CHUNK)],
                wsems[b])
        for b in range(nbuf):
            if wb_d[b] is not None:
                wb_d[b].wait()

    return gather


_sc_gather_half = (_make_gather(0), _make_gather(1))


_S_TSTEPS = 2 * _S_STEPS


@functools.partial(
    pl.kernel,
    mesh=_sc_mesh,
    out_type=jax.ShapeDtypeStruct((2, N_NODES, 16), _F32),
    scratch_types=[
        pltpu.VMEM((2, _S_CHUNK, 16), _F32),
        pltpu.VMEM((2, _S_IDXROWS, _S_IDXW), jnp.int32),
        pltpu.VMEM_SHARED((N_NODES, 16), _F32),
        pltpu.SemaphoreType.DMA,
        pltpu.SemaphoreType.DMA,
        pltpu.SemaphoreType.DMA,
        pltpu.SemaphoreType.DMA,
        pltpu.SemaphoreType.DMA,
    ],
    compiler_params=pltpu.CompilerParams(use_tc_tiling_on_sc=False),
)
def _sc_scatter_add(ma_hbm, mb_hbm, dst_hbm, zeros_hbm, out_hbm, m_v, idx_v,
                    acc, msem0, msem1, dsem0, dsem1, ssem):
    cid = lax.axis_index("c")
    sid = lax.axis_index("s")
    wid = sid * 2 + cid
    base0 = wid * _S_PER_W
    msems, dsems = (msem0, msem1), (dsem0, dsem1)

    @pl.when(sid == 0)
    def _():
        pltpu.sync_copy(zeros_hbm, acc)

    plsc.subcore_barrier()

    def _start_loads(i, b):
        if i < _S_STEPS:
            m_ref, edge0 = ma_hbm, 0
            base = base0 + i * _S_CHUNK
        else:
            m_ref, edge0 = mb_hbm, _HE
            base = base0 + (i - _S_STEPS) * _S_CHUNK
        md = pltpu.async_copy(m_ref.at[pl.ds(base, _S_CHUNK)], m_v.at[b],
                              msems[b])
        dd = pltpu.async_copy(
            dst_hbm.at[pl.ds((edge0 + base) // _S_IDXW, _S_IDXROWS)],
            idx_v.at[b], dsems[b])
        return md, dd

    loads = [None, None]
    loads[0] = _start_loads(0, 0)
    for i in range(_S_TSTEPS):
        b = i % 2
        loads[b][0].wait()
        loads[b][1].wait()
        if i + 1 < _S_TSTEPS:
            loads[1 - b] = _start_loads(i + 1, 1 - b)
        descs = []
        for j in range(_S_IDXROWS):
            descs.append(pltpu.async_copy(
                m_v.at[b].at[pl.ds(j * _S_IDXW, _S_IDXW)],
                acc.at[idx_v.at[b].at[j]],
                ssem, add=True))
        for d in descs:
            d.wait()

    plsc.subcore_barrier()

    @pl.when(sid == 0)
    def _():
        pltpu.sync_copy(acc, out_hbm.at[cid])


_NBLK = 2000
_EBLK = 2000
_EGRID = _EP // _EBLK
_HEP = _HE // 8
_HGRID = _HEP // _EBLK


def _dot(a, b):
    return jnp.dot(a, b, preferred_element_type=_F32)


def _full(shape):
    return pl.BlockSpec(shape, lambda i: (0,) * len(shape))


def _bd(w):
    return jnp.kron(jnp.eye(8, dtype=w.dtype), w)


def _tile8(b):
    return jnp.tile(b.reshape(1, -1), (1, 8))


def _node_enc_body(x_ref, w1_ref, w2_ref, o_ref):
    h = jnp.maximum(_dot(x_ref[...], w1_ref[...]), 0.0)
    o_ref[...] = jnp.maximum(_dot(h, w2_ref[...]), 0.0)


def _node_enc(x, w1, w2):
    return pl.pallas_call(
        _node_enc_body,
        grid=(N_NODES // _NBLK,),
        in_specs=[
            pl.BlockSpec((_NBLK, 128), lambda i: (i, 0)),
            _full((128, 64)),
            _full((64, 16)),
        ],
        out_specs=pl.BlockSpec((_NBLK, 16), lambda i: (i, 0)),
        out_shape=jax.ShapeDtypeStruct((N_NODES, 16), _F32),
    )(x, w1, w2)


def _edge_enc_body(ea_ref, w1_ref, w2_ref, o_ref):
    h = jnp.maximum(_dot(ea_ref[...], w1_ref[...]), 0.0)
    o_ref[...] = jnp.maximum(_dot(h, w2_ref[...]), 0.0)


def _edge_enc(ea_p, w1bd, w2bd):
    espec = pl.BlockSpec((_EBLK, 128), lambda i: (i, 0))
    return pl.pallas_call(
        _edge_enc_body,
        grid=(_EGRID,),
        in_specs=[espec, _full((128, 512)), _full((512, 128))],
        out_specs=espec,
        out_shape=jax.ShapeDtypeStruct((_EP, 128), _F32),
    )(ea_p, w1bd, w2bd)


def _edge_mlp_body(hs, hd, e, wa, wb, wc, b1, w2, b2, m_o, e_o):
    ev = e[...]
    pre = (_dot(hs[...], wa[...]) + _dot(hd[...], wb[...])
           + _dot(ev, wc[...]) + b1[...])
    m = _dot(jnp.maximum(pre, 0.0), w2[...]) + b2[...]
    m_o[...] = m
    e_o[...] = ALPHA * ev + (1.0 - ALPHA) * jnp.maximum(m, 0.0)


def _edge_mlp(g_p, e_p, eoff, wabd, wbbd, wcbd, b1t, w2bd, b2t):
    espec = pl.BlockSpec((_EBLK, 128), lambda i: (i, 0))
    return pl.pallas_call(
        _edge_mlp_body,
        grid=(_HGRID,),
        in_specs=[
            espec,
            pl.BlockSpec((_EBLK, 128), lambda i: (i + _HGRID, 0)),
            pl.BlockSpec((_EBLK, 128), lambda i: (i + eoff, 0)),
            _full((128, 512)), _full((128, 512)), _full((128, 512)),
            _full((1, 512)), _full((512, 128)), _full((1, 128)),
        ],
        out_specs=[espec, espec],
        out_shape=[
            jax.ShapeDtypeStruct((_HEP, 128), _F32),
            jax.ShapeDtypeStruct((_HEP, 128), _F32),
        ],
    )(g_p, g_p, e_p, wabd, wbbd, wcbd, b1t, w2bd, b2t)


def _edge_final_body(hs, hd, e, wa, wb, wc, b1, w2, b2,
                     v1, c1, v2, c2, v3, c3, o_ref):
    ev = e[...]
    pre = (_dot(hs[...], wa[...]) + _dot(hd[...], wb[...])
           + _dot(ev, wc[...]) + b1[...])
    m = _dot(jnp.maximum(pre, 0.0), w2[...]) + b2[...]
    e2 = ALPHA * ev + (1.0 - ALPHA) * jnp.maximum(m, 0.0)
    h1 = jnp.maximum(_dot(e2, v1[...]) + c1[...], 0.0)
    h2 = jnp.maximum(_dot(h1, v2[...]) + c2[...], 0.0)
    o_ref[...] = jax.nn.sigmoid(_dot(h2, v3[...]) + c3[...])


def _edge_final(g_p, e_p, eoff, wabd, wbbd, wcbd, b1t, w2bd, b2t,
                v1bd, c1t, v2bd, c2t, v3bd, c3t):
    espec = pl.BlockSpec((_EBLK, 128), lambda i: (i, 0))
    return pl.pallas_call(
        _edge_final_body,
        grid=(_HGRID,),
        in_specs=[
            espec,
            pl.BlockSpec((_EBLK, 128), lambda i: (i + _HGRID, 0)),
            pl.BlockSpec((_EBLK, 128), lambda i: (i + eoff, 0)),
            _full((128, 512)), _full((128, 512)), _full((128, 512)),
            _full((1, 512)), _full((512, 128)), _full((1, 128)),
            _full((128, 512)), _full((1, 512)),
            _full((512, 512)), _full((1, 512)),
            _full((512, 8)), _full((1, 8)),
        ],
        out_specs=pl.BlockSpec((_EBLK, 8), lambda i: (i, 0)),
        out_shape=jax.ShapeDtypeStruct((_HEP, 8), _F32),
    )(g_p, g_p, e_p, wabd, wbbd, wcbd, b1t, w2bd, b2t,
      v1bd, c1t, v2bd, c2t, v3bd, c3t)


def _node_mlp_body(h, p0, p1, wa, wb, b1, w2, b2, h_o):
    agg = p0[0] + p1[0]
    pre = _dot(h[...], wa[...]) + _dot(agg, wb[...]) + b1[...]
    hn = _dot(jnp.maximum(pre, 0.0), w2[...]) + b2[...]
    h_o[...] = ALPHA * h[...] + (1.0 - ALPHA) * jnp.maximum(hn, 0.0)


def _node_mlp(h, parts, wa, wb, b1, w2, b2):
    nspec = pl.BlockSpec((_NBLK, 16), lambda i: (i, 0))
    return pl.pallas_call(
        _node_mlp_body,
        grid=(N_NODES // _NBLK,),
        in_specs=[
            nspec,
            pl.BlockSpec((1, _NBLK, 16), lambda i: (0, i, 0)),
            pl.BlockSpec((1, _NBLK, 16), lambda i: (1, i, 0)),
            _full((16, 64)), _full((16, 64)),
            _full((1, 64)), _full((64, 16)), _full((1, 16)),
        ],
        out_specs=nspec,
        out_shape=jax.ShapeDtypeStruct((N_NODES, 16), _F32),
    )(h, parts, parts, wa, wb, b1, w2, b2)


def kernel(x, edge_index, edge_attr, params):
    idx_all = edge_index.reshape(-1)
    dst2d = edge_index[1].reshape(N_EDGES // _S_IDXW, _S_IDXW)
    zeros = jnp.zeros((N_NODES, 16), _F32)
    ea_p = edge_attr.reshape(-1).reshape(_EP, 128)

    h = _node_enc(x, params["node_enc"][0], params["node_enc"][1])
    e = _edge_enc(ea_p, _bd(params["edge_enc"][0]), _bd(params["edge_enc"][1]))
    e_half = (e, e)
    e_off = (0, _HGRID)

    out = None
    for l in range(3):
        p = params["resin"][l]
        w1, b1, w2, b2 = p["rel"]
        wabd, wbbd, wcbd = _bd(w1[0:16]), _bd(w1[16:32]), _bd(w1[32:48])
        b1t, b2t = _tile8(b1), _tile8(b2)
        w2bd = _bd(w2)
        rel = (wabd, wbbd, wcbd, b1t, w2bd, b2t)
        gA = _sc_gather_half[0](h, idx_all).reshape(2 * _HEP, 128)
        gB = _sc_gather_half[1](h, idx_all).reshape(2 * _HEP, 128)
        if l < 2:
            mA, eA = _edge_mlp(gA, e_half[0], e_off[0], *rel)
            mB, eB = _edge_mlp(gB, e_half[1], e_off[1], *rel)
            parts = _sc_scatter_add(mA.reshape(_HE, 16), mB.reshape(_HE, 16),
                                    dst2d, zeros)
            o1, ob1, o2, ob2 = p["obj"]
            h = _node_mlp(h, parts, o1[0:16], o1[16:32],
                          ob1.reshape(1, 64), o2, ob2.reshape(1, 16))
            e_half = (eA, eB)
            e_off = (0, 0)
        else:
            v1, c1, v2, c2, v3, c3 = params["W"]
            head = (_bd(v1), _tile8(c1), _bd(v2), _tile8(c2),
                    _bd(v3), _tile8(c3))
            outA = _edge_final(gA, e_half[0], e_off[0], *rel, *head)
            outB = _edge_final(gB, e_half[1], e_off[1], *rel, *head)
            out = jnp.concatenate([outA, outB], axis=0)
    return out.reshape(N_EDGES, 1)


# speedup vs baseline: 1.0014x; 1.0014x over previous
import functools

import jax
import jax.numpy as jnp
from jax import lax
from jax.experimental import pallas as pl
from jax.experimental.pallas import tpu as pltpu
from jax.experimental.pallas import tpu_sc as plsc

N_NODES = 10000
N_EDGES = 320000
ALPHA = 0.5

_F32 = jnp.float32
_EP = N_EDGES // 8


_NW = 32
_HE = N_EDGES // 2
_G_PER_W = 2 * _HE // _NW
_G_CHUNK = 1000
_G_STEPS = _G_PER_W // _G_CHUNK
_G_SRC_STEPS = _G_STEPS // 2

_S_PER_W = _HE // _NW
_S_CHUNK = 1000
_S_STEPS = _S_PER_W // _S_CHUNK
_S_IDXW = 100
_S_IDXROWS = _S_CHUNK // _S_IDXW

_sc_mesh = plsc.VectorSubcoreMesh(core_axis_name="c", subcore_axis_name="s")


def _make_gather(hx):
    half_w = _G_PER_W // 2

    @functools.partial(
        pl.kernel,
        mesh=_sc_mesh,
        out_type=jax.ShapeDtypeStruct((2 * _HE, 16), _F32),
        scratch_types=[
            pltpu.VMEM((2, _G_CHUNK), jnp.int32),
            pltpu.VMEM((2, _G_CHUNK, 16), _F32),
        ] + [pltpu.SemaphoreType.DMA] * 6,
        compiler_params=pltpu.CompilerParams(use_tc_tiling_on_sc=False),
    )
    def gather(table_hbm, idx_hbm, out_hbm, idx_v, rows_v,
               isem0, isem1, gsem0, gsem1, wsem0, wsem1):
        wid = lax.axis_index("s") * 2 + lax.axis_index("c")
        isems = (isem0, isem1)
        gsems = (gsem0, gsem1)
        wsems = (wsem0, wsem1)

        def bases(c):
            if c < _G_SRC_STEPS:
                off = wid * half_w + c * _G_CHUNK
                return hx * _HE + off, off
            off = wid * half_w + (c - _G_SRC_STEPS) * _G_CHUNK
            return N_EDGES + hx * _HE + off, _HE + off

        nbuf = 2
        idx_d = [None] * nbuf
        g_d = [None] * nbuf
        wb_d = [None] * nbuf
        for b in range(nbuf):
            idx_d[b] = pltpu.async_copy(
                idx_hbm.at[pl.ds(bases(b)[0], _G_CHUNK)], idx_v.at[b],
                isems[b])
        for b in range(nbuf - 1):
            idx_d[b].wait()
            g_d[b] = pltpu.async_copy(table_hbm.at[idx_v.at[b]],
                                      rows_v.at[b], gsems[b])
        for i in range(_G_STEPS):
            b = i % nbuf
            nxt = (i + nbuf - 1) % nbuf
            if i + nbuf - 1 < _G_STEPS:
                idx_d[nxt].wait()
                if wb_d[nxt] is not None:
                    wb_d[nxt].wait()
                g_d[nxt] = pltpu.async_copy(
                    table_hbm.at[idx_v.at[nxt]], rows_v.at[nxt], gsems[nxt])
            g_d[b].wait()
            if i + nbuf < _G_STEPS:
                idx_d[b] = pltpu.async_copy(
                    idx_hbm.at[pl.ds(bases(i + nbuf)[0], _G_CHUNK)],
                    idx_v.at[b], isems[b])
            wb_d[b] = pltpu.async_copy(
                rows_v.at[b], out_hbm.at[pl.ds(bases(i)[1], _G_CHUNK)],
                wsems[b])
        for b in range(nbuf):
            if wb_d[b] is not None:
                wb_d[b].wait()

    return gather


_sc_gather_half = (_make_gather(0), _make_gather(1))


_S_TSTEPS = 2 * _S_STEPS


@functools.partial(
    pl.kernel,
    mesh=_sc_mesh,
    out_type=jax.ShapeDtypeStruct((2, N_NODES, 16), _F32),
    scratch_types=[
        pltpu.VMEM((2, _S_CHUNK, 16), _F32),
        pltpu.VMEM((2, _S_IDXROWS, _S_IDXW), jnp.int32),
        pltpu.VMEM_SHARED((N_NODES, 16), _F32),
        pltpu.SemaphoreType.DMA,
        pltpu.SemaphoreType.DMA,
        pltpu.SemaphoreType.DMA,
        pltpu.SemaphoreType.DMA,
        pltpu.SemaphoreType.DMA,
    ],
    compiler_params=pltpu.CompilerParams(use_tc_tiling_on_sc=False),
)
def _sc_scatter_add(ma_hbm, mb_hbm, dst_hbm, zeros_hbm, out_hbm, m_v, idx_v,
                    acc, msem0, msem1, dsem0, dsem1, ssem):
    cid = lax.axis_index("c")
    sid = lax.axis_index("s")
    wid = sid * 2 + cid
    base0 = wid * _S_PER_W
    msems, dsems = (msem0, msem1), (dsem0, dsem1)

    @pl.when(sid == 0)
    def _():
        pltpu.sync_copy(zeros_hbm, acc)

    plsc.subcore_barrier()

    def _start_loads(i, b):
        if i < _S_STEPS:
            m_ref, edge0 = ma_hbm, 0
            base = base0 + i * _S_CHUNK
        else:
            m_ref, edge0 = mb_hbm, _HE
            base = base0 + (i - _S_STEPS) * _S_CHUNK
        md = pltpu.async_copy(m_ref.at[pl.ds(base, _S_CHUNK)], m_v.at[b],
                              msems[b])
        dd = pltpu.async_copy(
            dst_hbm.at[pl.ds((edge0 + base) // _S_IDXW, _S_IDXROWS)],
            idx_v.at[b], dsems[b])
        return md, dd

    loads = [None, None]
    loads[0] = _start_loads(0, 0)
    for i in range(_S_TSTEPS):
        b = i % 2
        loads[b][0].wait()
        loads[b][1].wait()
        if i + 1 < _S_TSTEPS:
            loads[1 - b] = _start_loads(i + 1, 1 - b)
        descs = []
        for j in range(_S_IDXROWS):
            descs.append(pltpu.async_copy(
                m_v.at[b].at[pl.ds(j * _S_IDXW, _S_IDXW)],
                acc.at[idx_v.at[b].at[j]],
                ssem, add=True))
        for d in descs:
            d.wait()

    plsc.subcore_barrier()

    @pl.when(sid == 0)
    def _():
        pltpu.sync_copy(acc, out_hbm.at[cid])


_NBLK = 2000
_EBLK = 2000
_EGRID = _EP // _EBLK
_HEP = _HE // 8
_HGRID = _HEP // _EBLK


def _dot(a, b):
    return jnp.dot(a, b, preferred_element_type=_F32)


def _full(shape):
    return pl.BlockSpec(shape, lambda i: (0,) * len(shape))


def _bd(w):
    return jnp.kron(jnp.eye(8, dtype=w.dtype), w)


def _tile8(b):
    return jnp.tile(b.reshape(1, -1), (1, 8))


def _node_enc_body(x_ref, w1_ref, w2_ref, o_ref):
    h = jnp.maximum(_dot(x_ref[...], w1_ref[...]), 0.0)
    o_ref[...] = jnp.maximum(_dot(h, w2_ref[...]), 0.0)


def _node_enc(x, w1, w2):
    return pl.pallas_call(
        _node_enc_body,
        grid=(N_NODES // _NBLK,),
        in_specs=[
            pl.BlockSpec((_NBLK, 128), lambda i: (i, 0)),
            _full((128, 64)),
            _full((64, 16)),
        ],
        out_specs=pl.BlockSpec((_NBLK, 16), lambda i: (i, 0)),
        out_shape=jax.ShapeDtypeStruct((N_NODES, 16), _F32),
    )(x, w1, w2)


def _edge_enc_body(ea_ref, w1_ref, w2_ref, o_ref):
    h = jnp.maximum(_dot(ea_ref[...], w1_ref[...]), 0.0)
    o_ref[...] = jnp.maximum(_dot(h, w2_ref[...]), 0.0)


def _edge_enc(ea_p, w1bd, w2bd):
    espec = pl.BlockSpec((_EBLK, 128), lambda i: (i, 0))
    return pl.pallas_call(
        _edge_enc_body,
        grid=(_EGRID,),
        in_specs=[espec, _full((128, 512)), _full((512, 128))],
        out_specs=espec,
        out_shape=jax.ShapeDtypeStruct((_EP, 128), _F32),
    )(ea_p, w1bd, w2bd)


def _edge_mlp_body(hs, hd, e, wa, wb, wc, b1, w2, b2, m_o, e_o):
    ev = e[...]
    pre = (_dot(hs[...], wa[...]) + _dot(hd[...], wb[...])
           + _dot(ev, wc[...]) + b1[...])
    m = _dot(jnp.maximum(pre, 0.0), w2[...]) + b2[...]
    m_o[...] = m
    e_o[...] = ALPHA * ev + (1.0 - ALPHA) * jnp.maximum(m, 0.0)


def _edge_mlp(g_p, e_p, eoff, wabd, wbbd, wcbd, b1t, w2bd, b2t):
    espec = pl.BlockSpec((_EBLK, 128), lambda i: (i, 0))
    return pl.pallas_call(
        _edge_mlp_body,
        grid=(_HGRID,),
        in_specs=[
            espec,
            pl.BlockSpec((_EBLK, 128), lambda i: (i + _HGRID, 0)),
            pl.BlockSpec((_EBLK, 128), lambda i: (i + eoff, 0)),
            _full((128, 512)), _full((128, 512)), _full((128, 512)),
            _full((1, 512)), _full((512, 128)), _full((1, 128)),
        ],
        out_specs=[espec, espec],
        out_shape=[
            jax.ShapeDtypeStruct((_HEP, 128), _F32),
            jax.ShapeDtypeStruct((_HEP, 128), _F32),
        ],
    )(g_p, g_p, e_p, wabd, wbbd, wcbd, b1t, w2bd, b2t)


def _edge_final_body(hs, hd, e, wa, wb, wc, b1, w2, b2,
                     v1, c1, v2, c2, v3, c3, o_ref):
    ev = e[...]
    pre = (_dot(hs[...], wa[...]) + _dot(hd[...], wb[...])
           + _dot(ev, wc[...]) + b1[...])
    m = _dot(jnp.maximum(pre, 0.0), w2[...]) + b2[...]
    e2 = ALPHA * ev + (1.0 - ALPHA) * jnp.maximum(m, 0.0)
    h1 = jnp.maximum(_dot(e2, v1[...]) + c1[...], 0.0)
    h2 = jnp.maximum(_dot(h1, v2[...]) + c2[...], 0.0)
    o_ref[...] = jax.nn.sigmoid(_dot(h2, v3[...]) + c3[...])


def _edge_final(g_p, e_p, eoff, wabd, wbbd, wcbd, b1t, w2bd, b2t,
                v1bd, c1t, v2bd, c2t, v3bd, c3t):
    espec = pl.BlockSpec((_EBLK, 128), lambda i: (i, 0))
    return pl.pallas_call(
        _edge_final_body,
        grid=(_HGRID,),
        in_specs=[
            espec,
            pl.BlockSpec((_EBLK, 128), lambda i: (i + _HGRID, 0)),
            pl.BlockSpec((_EBLK, 128), lambda i: (i + eoff, 0)),
            _full((128, 512)), _full((128, 512)), _full((128, 512)),
            _full((1, 512)), _full((512, 128)), _full((1, 128)),
            _full((128, 512)), _full((1, 512)),
            _full((512, 512)), _full((1, 512)),
            _full((512, 8)), _full((1, 8)),
        ],
        out_specs=pl.BlockSpec((_EBLK, 8), lambda i: (i, 0)),
        out_shape=jax.ShapeDtypeStruct((_HEP, 8), _F32),
    )(g_p, g_p, e_p, wabd, wbbd, wcbd, b1t, w2bd, b2t,
      v1bd, c1t, v2bd, c2t, v3bd, c3t)


def _node_mlp_body(h, p0, p1, wa, wb, b1, w2, b2, h_o):
    agg = p0[0] + p1[0]
    pre = _dot(h[...], wa[...]) + _dot(agg, wb[...]) + b1[...]
    hn = _dot(jnp.maximum(pre, 0.0), w2[...]) + b2[...]
    h_o[...] = ALPHA * h[...] + (1.0 - ALPHA) * jnp.maximum(hn, 0.0)


def _node_mlp(h, parts, wa, wb, b1, w2, b2):
    nspec = pl.BlockSpec((_NBLK, 16), lambda i: (i, 0))
    return pl.pallas_call(
        _node_mlp_body,
        grid=(N_NODES // _NBLK,),
        in_specs=[
            nspec,
            pl.BlockSpec((1, _NBLK, 16), lambda i: (0, i, 0)),
            pl.BlockSpec((1, _NBLK, 16), lambda i: (1, i, 0)),
            _full((16, 64)), _full((16, 64)),
            _full((1, 64)), _full((64, 16)), _full((1, 16)),
        ],
        out_specs=nspec,
        out_shape=jax.ShapeDtypeStruct((N_NODES, 16), _F32),
    )(h, parts, parts, wa, wb, b1, w2, b2)


def kernel(x, edge_index, edge_attr, params):
    idx_all = edge_index.reshape(-1)
    dst2d = edge_index[1].reshape(N_EDGES // _S_IDXW, _S_IDXW)
    zeros = jnp.zeros((N_NODES, 16), _F32)
    ea_p = edge_attr.reshape(-1).reshape(_EP, 128)

    h = _node_enc(x, params["node_enc"][0], params["node_enc"][1])
    e = _edge_enc(ea_p, _bd(params["edge_enc"][0]), _bd(params["edge_enc"][1]))
    e_half = (e, e)
    e_off = (0, _HGRID)

    out = None
    for l in range(3):
        p = params["resin"][l]
        w1, b1, w2, b2 = p["rel"]
        wabd, wbbd, wcbd = _bd(w1[0:16]), _bd(w1[16:32]), _bd(w1[32:48])
        b1t, b2t = _tile8(b1), _tile8(b2)
        w2bd = _bd(w2)
        rel = (wabd, wbbd, wcbd, b1t, w2bd, b2t)
        gA = _sc_gather_half[0](h, idx_all).reshape(2 * _HEP, 128)
        gB = _sc_gather_half[1](h, idx_all).reshape(2 * _HEP, 128)
        if l < 2:
            mA, eA = _edge_mlp(gA, e_half[0], e_off[0], *rel)
            mB, eB = _edge_mlp(gB, e_half[1], e_off[1], *rel)
            parts = _sc_scatter_add(mA.reshape(_HE, 16), mB.reshape(_HE, 16),
                                    dst2d, zeros)
            o1, ob1, o2, ob2 = p["obj"]
            h = _node_mlp(h, parts, o1[0:16], o1[16:32],
                          ob1.reshape(1, 64), o2, ob2.reshape(1, 16))
            e_half = (eA, eB)
            e_off = (0, 0)
        else:
            v1, c1, v2, c2, v3, c3 = params["W"]
            head = (_bd(v1), _tile8(c1), _bd(v2), _tile8(c2),
                    _bd(v3), _tile8(c3))
            outA = _edge_final(gA, e_half[0], e_off[0], *rel, *head)
            outB = _edge_final(gB, e_half[1], e_off[1], *rel, *head)
            out = jnp.concatenate([outA, outB], axis=0)
    return out.reshape(N_EDGES, 1)


# speedup vs baseline: 1.0076x; 1.0062x over previous
import functools

import jax
import jax.numpy as jnp
from jax import lax
from jax.experimental import pallas as pl
from jax.experimental.pallas import tpu as pltpu
from jax.experimental.pallas import tpu_sc as plsc

N_NODES = 10000
N_EDGES = 320000
ALPHA = 0.5

_F32 = jnp.float32
_EP = N_EDGES // 8


_NW = 32
_HE = N_EDGES // 2
_G_PER_W = 2 * _HE // _NW
_G_CHUNK = 1000
_G_STEPS = _G_PER_W // _G_CHUNK
_G_SRC_STEPS = _G_STEPS // 2

_S_PER_W = _HE // _NW
_S_CHUNK = 1000
_S_STEPS = _S_PER_W // _S_CHUNK
_S_IDXW = 100
_S_IDXROWS = _S_CHUNK // _S_IDXW

_sc_mesh = plsc.VectorSubcoreMesh(core_axis_name="c", subcore_axis_name="s")


def _make_gather(hx):
    half_w = _G_PER_W // 2

    @functools.partial(
        pl.kernel,
        mesh=_sc_mesh,
        out_type=jax.ShapeDtypeStruct((2 * _HE, 16), _F32),
        scratch_types=[
            pltpu.VMEM((2, _G_CHUNK), jnp.int32),
            pltpu.VMEM((2, _G_CHUNK, 16), _F32),
        ] + [pltpu.SemaphoreType.DMA] * 6,
        compiler_params=pltpu.CompilerParams(use_tc_tiling_on_sc=False),
    )
    def gather(table_hbm, idx_hbm, out_hbm, idx_v, rows_v,
               isem0, isem1, gsem0, gsem1, wsem0, wsem1):
        wid = lax.axis_index("s") * 2 + lax.axis_index("c")
        isems = (isem0, isem1)
        gsems = (gsem0, gsem1)
        wsems = (wsem0, wsem1)

        def bases(c):
            if c < _G_SRC_STEPS:
                off = wid * half_w + c * _G_CHUNK
                return hx * _HE + off, off
            off = wid * half_w + (c - _G_SRC_STEPS) * _G_CHUNK
            return N_EDGES + hx * _HE + off, _HE + off

        nbuf = 2
        idx_d = [None] * nbuf
        g_d = [None] * nbuf
        wb_d = [None] * nbuf
        for b in range(nbuf):
            idx_d[b] = pltpu.async_copy(
                idx_hbm.at[pl.ds(bases(b)[0], _G_CHUNK)], idx_v.at[b],
                isems[b])
        for b in range(nbuf - 1):
            idx_d[b].wait()
            g_d[b] = pltpu.async_copy(table_hbm.at[idx_v.at[b]],
                                      rows_v.at[b], gsems[b])
        for i in range(_G_STEPS):
            b = i % nbuf
            nxt = (i + nbuf - 1) % nbuf
            if i + nbuf - 1 < _G_STEPS:
                idx_d[nxt].wait()
                if wb_d[nxt] is not None:
                    wb_d[nxt].wait()
                g_d[nxt] = pltpu.async_copy(
                    table_hbm.at[idx_v.at[nxt]], rows_v.at[nxt], gsems[nxt])
            g_d[b].wait()
            if i + nbuf < _G_STEPS:
                idx_d[b] = pltpu.async_copy(
                    idx_hbm.at[pl.ds(bases(i + nbuf)[0], _G_CHUNK)],
                    idx_v.at[b], isems[b])
            wb_d[b] = pltpu.async_copy(
                rows_v.at[b], out_hbm.at[pl.ds(bases(i)[1], _G_CHUNK)],
                wsems[b])
        for b in range(nbuf):
            if wb_d[b] is not None:
                wb_d[b].wait()

    return gather


_sc_gather_half = (_make_gather(0), _make_gather(1))


def _make_scatter(hx):
    @functools.partial(
        pl.kernel,
        mesh=_sc_mesh,
        out_type=jax.ShapeDtypeStruct((2, N_NODES, 16), _F32),
        scratch_types=[
            pltpu.VMEM((2, _S_CHUNK, 16), _F32),
            pltpu.VMEM((2, _S_IDXROWS, _S_IDXW), jnp.int32),
            pltpu.VMEM_SHARED((N_NODES, 16), _F32),
            pltpu.SemaphoreType.DMA,
            pltpu.SemaphoreType.DMA,
            pltpu.SemaphoreType.DMA,
            pltpu.SemaphoreType.DMA,
            pltpu.SemaphoreType.DMA,
        ],
        compiler_params=pltpu.CompilerParams(use_tc_tiling_on_sc=False),
    )
    def scatter(m_hbm, dst_hbm, zeros_hbm, out_hbm, m_v, idx_v, acc,
                msem0, msem1, dsem0, dsem1, ssem):
        cid = lax.axis_index("c")
        sid = lax.axis_index("s")
        wid = sid * 2 + cid
        base0 = wid * _S_PER_W
        msems, dsems = (msem0, msem1), (dsem0, dsem1)

        @pl.when(sid == 0)
        def _():
            pltpu.sync_copy(zeros_hbm, acc)

        plsc.subcore_barrier()

        def _start_loads(i, b):
            base = base0 + i * _S_CHUNK
            md = pltpu.async_copy(m_hbm.at[pl.ds(base, _S_CHUNK)], m_v.at[b],
                                  msems[b])
            dd = pltpu.async_copy(
                dst_hbm.at[pl.ds((hx * _HE + base) // _S_IDXW, _S_IDXROWS)],
                idx_v.at[b], dsems[b])
            return md, dd

        loads = [None, None]
        loads[0] = _start_loads(0, 0)
        for i in range(_S_STEPS):
            b = i % 2
            loads[b][0].wait()
            loads[b][1].wait()
            if i + 1 < _S_STEPS:
                loads[1 - b] = _start_loads(i + 1, 1 - b)
            descs = []
            for j in range(_S_IDXROWS):
                descs.append(pltpu.async_copy(
                    m_v.at[b].at[pl.ds(j * _S_IDXW, _S_IDXW)],
                    acc.at[idx_v.at[b].at[j]],
                    ssem, add=True))
            for d in descs:
                d.wait()

        plsc.subcore_barrier()

        @pl.when(sid == 0)
        def _():
            pltpu.sync_copy(acc, out_hbm.at[cid])

    return scatter


_sc_scatter_half = (_make_scatter(0), _make_scatter(1))


_NBLK = 2000
_EBLK = 2000
_EGRID = _EP // _EBLK
_HEP = _HE // 8
_HGRID = _HEP // _EBLK


def _dot(a, b):
    return jnp.dot(a, b, preferred_element_type=_F32)


def _full(shape):
    return pl.BlockSpec(shape, lambda i: (0,) * len(shape))


def _bd(w):
    return jnp.kron(jnp.eye(8, dtype=w.dtype), w)


def _tile8(b):
    return jnp.tile(b.reshape(1, -1), (1, 8))


def _node_enc_body(x_ref, w1_ref, w2_ref, o_ref):
    h = jnp.maximum(_dot(x_ref[...], w1_ref[...]), 0.0)
    o_ref[...] = jnp.maximum(_dot(h, w2_ref[...]), 0.0)


def _node_enc(x, w1, w2):
    return pl.pallas_call(
        _node_enc_body,
        grid=(N_NODES // _NBLK,),
        in_specs=[
            pl.BlockSpec((_NBLK, 128), lambda i: (i, 0)),
            _full((128, 64)),
            _full((64, 16)),
        ],
        out_specs=pl.BlockSpec((_NBLK, 16), lambda i: (i, 0)),
        out_shape=jax.ShapeDtypeStruct((N_NODES, 16), _F32),
    )(x, w1, w2)


def _edge_enc_body(ea_ref, w1_ref, w2_ref, o_ref):
    h = jnp.maximum(_dot(ea_ref[...], w1_ref[...]), 0.0)
    o_ref[...] = jnp.maximum(_dot(h, w2_ref[...]), 0.0)


def _edge_enc(ea_p, w1bd, w2bd):
    espec = pl.BlockSpec((_EBLK, 128), lambda i: (i, 0))
    return pl.pallas_call(
        _edge_enc_body,
        grid=(_EGRID,),
        in_specs=[espec, _full((128, 512)), _full((512, 128))],
        out_specs=espec,
        out_shape=jax.ShapeDtypeStruct((_EP, 128), _F32),
    )(ea_p, w1bd, w2bd)


def _edge_mlp_body(hs, hd, e, wa, wb, wc, b1, w2, b2, m_o, e_o):
    ev = e[...]
    pre = (_dot(hs[...], wa[...]) + _dot(hd[...], wb[...])
           + _dot(ev, wc[...]) + b1[...])
    m = _dot(jnp.maximum(pre, 0.0), w2[...]) + b2[...]
    m_o[...] = m
    e_o[...] = ALPHA * ev + (1.0 - ALPHA) * jnp.maximum(m, 0.0)


def _edge_mlp(g_p, e_p, eoff, wabd, wbbd, wcbd, b1t, w2bd, b2t):
    espec = pl.BlockSpec((_EBLK, 128), lambda i: (i, 0))
    return pl.pallas_call(
        _edge_mlp_body,
        grid=(_HGRID,),
        in_specs=[
            espec,
            pl.BlockSpec((_EBLK, 128), lambda i: (i + _HGRID, 0)),
            pl.BlockSpec((_EBLK, 128), lambda i: (i + eoff, 0)),
            _full((128, 512)), _full((128, 512)), _full((128, 512)),
            _full((1, 512)), _full((512, 128)), _full((1, 128)),
        ],
        out_specs=[espec, espec],
        out_shape=[
            jax.ShapeDtypeStruct((_HEP, 128), _F32),
            jax.ShapeDtypeStruct((_HEP, 128), _F32),
        ],
    )(g_p, g_p, e_p, wabd, wbbd, wcbd, b1t, w2bd, b2t)


def _edge_final_body(hs, hd, e, wa, wb, wc, b1, w2, b2,
                     v1, c1, v2, c2, v3, c3, o_ref):
    ev = e[...]
    pre = (_dot(hs[...], wa[...]) + _dot(hd[...], wb[...])
           + _dot(ev, wc[...]) + b1[...])
    m = _dot(jnp.maximum(pre, 0.0), w2[...]) + b2[...]
    e2 = ALPHA * ev + (1.0 - ALPHA) * jnp.maximum(m, 0.0)
    h1 = jnp.maximum(_dot(e2, v1[...]) + c1[...], 0.0)
    h2 = jnp.maximum(_dot(h1, v2[...]) + c2[...], 0.0)
    o_ref[...] = jax.nn.sigmoid(_dot(h2, v3[...]) + c3[...])


def _edge_final(g_p, e_p, eoff, wabd, wbbd, wcbd, b1t, w2bd, b2t,
                v1bd, c1t, v2bd, c2t, v3bd, c3t):
    espec = pl.BlockSpec((_EBLK, 128), lambda i: (i, 0))
    return pl.pallas_call(
        _edge_final_body,
        grid=(_HGRID,),
        in_specs=[
            espec,
            pl.BlockSpec((_EBLK, 128), lambda i: (i + _HGRID, 0)),
            pl.BlockSpec((_EBLK, 128), lambda i: (i + eoff, 0)),
            _full((128, 512)), _full((128, 512)), _full((128, 512)),
            _full((1, 512)), _full((512, 128)), _full((1, 128)),
            _full((128, 512)), _full((1, 512)),
            _full((512, 512)), _full((1, 512)),
            _full((512, 8)), _full((1, 8)),
        ],
        out_specs=pl.BlockSpec((_EBLK, 8), lambda i: (i, 0)),
        out_shape=jax.ShapeDtypeStruct((_HEP, 8), _F32),
    )(g_p, g_p, e_p, wabd, wbbd, wcbd, b1t, w2bd, b2t,
      v1bd, c1t, v2bd, c2t, v3bd, c3t)


def _node_mlp_body(h, p0, p1, wa, wb, b1, w2, b2, h_o):
    agg = p0[0] + p1[0]
    pre = _dot(h[...], wa[...]) + _dot(agg, wb[...]) + b1[...]
    hn = _dot(jnp.maximum(pre, 0.0), w2[...]) + b2[...]
    h_o[...] = ALPHA * h[...] + (1.0 - ALPHA) * jnp.maximum(hn, 0.0)


def _node_mlp4_body(h, p0, p1, p2, p3, wa, wb, b1, w2, b2, h_o):
    agg = (p0[0] + p1[0]) + (p2[0] + p3[0])
    pre = _dot(h[...], wa[...]) + _dot(agg, wb[...]) + b1[...]
    hn = _dot(jnp.maximum(pre, 0.0), w2[...]) + b2[...]
    h_o[...] = ALPHA * h[...] + (1.0 - ALPHA) * jnp.maximum(hn, 0.0)


def _node_mlp_4(h, parts_a, parts_b, wa, wb, b1, w2, b2):
    nspec = pl.BlockSpec((_NBLK, 16), lambda i: (i, 0))
    p0spec = pl.BlockSpec((1, _NBLK, 16), lambda i: (0, i, 0))
    p1spec = pl.BlockSpec((1, _NBLK, 16), lambda i: (1, i, 0))
    return pl.pallas_call(
        _node_mlp4_body,
        grid=(N_NODES // _NBLK,),
        in_specs=[
            nspec, p0spec, p1spec, p0spec, p1spec,
            _full((16, 64)), _full((16, 64)),
            _full((1, 64)), _full((64, 16)), _full((1, 16)),
        ],
        out_specs=nspec,
        out_shape=jax.ShapeDtypeStruct((N_NODES, 16), _F32),
    )(h, parts_a, parts_a, parts_b, parts_b, wa, wb, b1, w2, b2)


def kernel(x, edge_index, edge_attr, params):
    idx_all = edge_index.reshape(-1)
    dst2d = edge_index[1].reshape(N_EDGES // _S_IDXW, _S_IDXW)
    zeros = jnp.zeros((N_NODES, 16), _F32)
    ea_p = edge_attr.reshape(-1).reshape(_EP, 128)

    h = _node_enc(x, params["node_enc"][0], params["node_enc"][1])
    e = _edge_enc(ea_p, _bd(params["edge_enc"][0]), _bd(params["edge_enc"][1]))
    e_half = (e, e)
    e_off = (0, _HGRID)

    out = None
    for l in range(3):
        p = params["resin"][l]
        w1, b1, w2, b2 = p["rel"]
        wabd, wbbd, wcbd = _bd(w1[0:16]), _bd(w1[16:32]), _bd(w1[32:48])
        b1t, b2t = _tile8(b1), _tile8(b2)
        w2bd = _bd(w2)
        rel = (wabd, wbbd, wcbd, b1t, w2bd, b2t)
        gA = _sc_gather_half[0](h, idx_all).reshape(2 * _HEP, 128)
        gB = _sc_gather_half[1](h, idx_all).reshape(2 * _HEP, 128)
        if l < 2:
            mA, eA = _edge_mlp(gA, e_half[0], e_off[0], *rel)
            mB, eB = _edge_mlp(gB, e_half[1], e_off[1], *rel)
            pA = _sc_scatter_half[0](mA.reshape(_HE, 16), dst2d, zeros)
            pB = _sc_scatter_half[1](mB.reshape(_HE, 16), dst2d, zeros)
            o1, ob1, o2, ob2 = p["obj"]
            h = _node_mlp_4(h, pA, pB, o1[0:16], o1[16:32],
                            ob1.reshape(1, 64), o2, ob2.reshape(1, 16))
            e_half = (eA, eB)
            e_off = (0, 0)
        else:
            v1, c1, v2, c2, v3, c3 = params["W"]
            head = (_bd(v1), _tile8(c1), _bd(v2), _tile8(c2),
                    _bd(v3), _tile8(c3))
            outA = _edge_final(gA, e_half[0], e_off[0], *rel, *head)
            outB = _edge_final(gB, e_half[1], e_off[1], *rel, *head)
            out = jnp.concatenate([outA, outB], axis=0)
    return out.reshape(N_EDGES, 1)
